# baseline (speedup 1.0000x reference)
import functools
import math

import jax
import jax.numpy as jnp
from jax import lax
from jax.experimental import pallas as pl
from jax.experimental.pallas import tpu as pltpu
from jax.experimental.pallas import tpu_sc as plsc

_LANES = 16


@functools.lru_cache(maxsize=None)
def _make_emb_kernel(batch: int, d: int, num_workers: int, chunk: int):
    assert batch % num_workers == 0
    b_per_w = batch // num_workers
    assert b_per_w % chunk == 0
    n_chunks = b_per_w // chunk
    scale = math.sqrt(d)
    mesh = plsc.VectorSubcoreMesh(core_axis_name="c", subcore_axis_name="s")

    @functools.partial(
        pl.kernel,
        mesh=mesh,
        compiler_params=pltpu.CompilerParams(use_tc_tiling_on_sc=False),
        out_type=jax.ShapeDtypeStruct((batch, d), jnp.float32),
        scratch_types=[
            pltpu.VMEM((b_per_w,), jnp.int32),
            pltpu.VMEM((chunk, d), jnp.float32),
            pltpu.SemaphoreType.DMA,
        ],
    )
    def emb(table_hbm, idx_hbm, out_hbm, idx_v, rows_v, sem):
        wid = lax.axis_index("s") * 2 + lax.axis_index("c")
        base = wid * b_per_w
        pltpu.sync_copy(idx_hbm.at[pl.ds(base, b_per_w)], idx_v)

        def do_chunk(j, carry):
            pltpu.async_copy(
                table_hbm.at[idx_v.at[pl.ds(j * chunk, chunk)]], rows_v, sem
            ).wait()

            def scale_row(r, carry2):
                for p in range(d // _LANES):
                    sl = pl.ds(p * _LANES, _LANES)
                    rows_v[r, sl] = rows_v[r, sl] * scale
                return carry2

            lax.fori_loop(0, chunk, scale_row, 0, unroll=4)
            pltpu.sync_copy(rows_v, out_hbm.at[pl.ds(base + j * chunk, chunk)])
            return carry

        lax.fori_loop(0, n_chunks, do_chunk, 0)

    return emb


def kernel(x, table):
    b0, b1 = x.shape
    v, d = table.shape
    batch = b0 * b1
    idx = x.reshape(batch).astype(jnp.int32)
    emb = _make_emb_kernel(batch, d, 32, 512)
    out = emb(table, idx)
    return out.reshape(b0, b1, d)


# speedup vs baseline: 1.0707x; 1.0707x over previous
import functools
import math

import jax
import jax.numpy as jnp
from jax import lax
from jax.experimental import pallas as pl
from jax.experimental.pallas import tpu as pltpu
from jax.experimental.pallas import tpu_sc as plsc

_LANES = 16
_NBUF = 4


@functools.lru_cache(maxsize=None)
def _make_emb_kernel(batch: int, d: int, num_workers: int, chunk: int):
    assert batch % num_workers == 0
    b_per_w = batch // num_workers
    assert b_per_w % chunk == 0
    n_chunks = b_per_w // chunk
    assert n_chunks % _NBUF == 0 and n_chunks >= 2 * _NBUF
    scale = math.sqrt(d)
    mesh = plsc.VectorSubcoreMesh(core_axis_name="c", subcore_axis_name="s")

    @functools.partial(
        pl.kernel,
        mesh=mesh,
        compiler_params=pltpu.CompilerParams(use_tc_tiling_on_sc=False),
        out_type=jax.ShapeDtypeStruct((batch, d), jnp.float32),
        scratch_types=[
            pltpu.VMEM((b_per_w,), jnp.int32),
            pltpu.VMEM((_NBUF, chunk, d), jnp.float32),
            [pltpu.SemaphoreType.DMA] * _NBUF,
            [pltpu.SemaphoreType.DMA] * _NBUF,
        ],
    )
    def emb(table_hbm, idx_hbm, out_hbm, idx_v, rows_v, gsems, wsems):
        wid = lax.axis_index("s") * 2 + lax.axis_index("c")
        base = wid * b_per_w
        pltpu.sync_copy(idx_hbm.at[pl.ds(base, b_per_w)], idx_v)

        def gather_desc(j, b):
            return pltpu.make_async_copy(
                table_hbm.at[idx_v.at[pl.ds(j * chunk, chunk)]],
                rows_v.at[b],
                gsems[b],
            )

        def write_desc(j, b):
            return pltpu.make_async_copy(
                rows_v.at[b],
                out_hbm.at[pl.ds(base + j * chunk, chunk)],
                wsems[b],
            )

        def scale_buf(b):
            def scale_row(r, carry):
                for p in range(d // _LANES):
                    sl = pl.ds(p * _LANES, _LANES)
                    rows_v[b, r, sl] = rows_v[b, r, sl] * scale
                return carry

            lax.fori_loop(0, chunk, scale_row, 0, unroll=8)

        gather_desc(0, 0).start()
        gather_desc(1, 1).start()

        def outer(p, carry):
            for b in range(_NBUF):
                j = p * _NBUF + b
                b2 = (b + 2) % _NBUF
                @pl.when(j >= 2)
                def _wait_prev():
                    write_desc(j - 2, b2).wait()

                @pl.when(j + 2 < n_chunks)
                def _refill():
                    gather_desc(j + 2, b2).start()

                gather_desc(j, b).wait()
                scale_buf(b)
                write_desc(j, b).start()
            return carry

        lax.fori_loop(0, n_chunks // _NBUF, outer, 0)
        write_desc(n_chunks - 2, (n_chunks - 2) % _NBUF).wait()
        write_desc(n_chunks - 1, (n_chunks - 1) % _NBUF).wait()

    return emb


def kernel(x, table):
    b0, b1 = x.shape
    v, d = table.shape
    batch = b0 * b1
    idx = x.reshape(batch).astype(jnp.int32)
    emb = _make_emb_kernel(batch, d, 32, 256)
    out = emb(table, idx)
    return out.reshape(b0, b1, d)
